# Initial kernel scaffold; baseline (speedup 1.0000x reference)
import jax
import jax.numpy as jnp
from jax.experimental import pallas as pl


def kernel(x):
    raise NotImplementedError("write your pallas kernel here")


# speedup vs baseline: 21.1780x; 21.1780x over previous
import functools

import jax
import jax.numpy as jnp
from jax.experimental import pallas as pl
from jax.experimental.pallas import tpu as pltpu


def _select_mask_body(k, x_ref, o_ref, ku_ref):
    x = x_ref[...]
    u = jax.lax.bitcast_convert_type(x, jnp.uint32)
    flip = jnp.where(
        (u >> 31) != 0, jnp.uint32(0xFFFFFFFF), jnp.uint32(0x80000000)
    )
    ku_ref[...] = u ^ flip

    g = x.shape[0]
    prefix = jnp.zeros((g, 1, 1), jnp.uint32)
    kf = jnp.float32(k)
    for s in range(31, -1, -1):
        cand = prefix | (jnp.uint32(1) << s)
        cnt = jnp.sum(
            (ku_ref[...] >= cand).astype(jnp.float32),
            axis=(1, 2),
            keepdims=True,
        )
        prefix = jnp.where(cnt >= kf, cand, prefix)

    unflip = jnp.where(
        (prefix >> 31) != 0, jnp.uint32(0x80000000), jnp.uint32(0xFFFFFFFF)
    )
    thresh = jax.lax.bitcast_convert_type(prefix ^ unflip, jnp.float32)
    o_ref[...] = x * (x >= thresh).astype(x.dtype)


def kernel(x):
    b, c, h, w = x.shape
    n = c * h * w
    k = int(0.1 * n)
    assert n % 128 == 0
    r = n // 128
    g = 4
    assert b % g == 0
    xr = x.reshape(b, r, 128)

    out = pl.pallas_call(
        functools.partial(_select_mask_body, k),
        grid=(b // g,),
        in_specs=[pl.BlockSpec((g, r, 128), lambda i: (i, 0, 0))],
        out_specs=pl.BlockSpec((g, r, 128), lambda i: (i, 0, 0)),
        out_shape=jax.ShapeDtypeStruct((b, r, 128), jnp.float32),
        scratch_shapes=[pltpu.VMEM((g, r, 128), jnp.uint32)],
    )(xr)
    return out.reshape(x.shape)
